# baseline (speedup 1.0000x reference)
import functools

import jax
import jax.numpy as jnp
import numpy as np
from jax import lax
from jax.experimental import pallas as pl
from jax.experimental.pallas import tpu as pltpu
from jax.experimental.pallas import tpu_sc as plsc

N = 10000
NPAD = 10016
B = 250
E = 390000
EPAD = 393216
HID = 64
NC = 2
NS = 16
NW = NC * NS
EPW = EPAD // NW
CH_G = 256
CH_S = 2048
NOUT = 10112
SIGMA = 25.0
LOG_SIGMA = 3.2188758248682006

_MESH = plsc.VectorSubcoreMesh(core_axis_name="c", subcore_axis_name="s",
                               num_cores=NC, num_subcores=NS)

_h16 = np.ravel(np.column_stack([np.arange(16), np.arange(16) + 16]))
_HPERM = np.concatenate([_h16, _h16 + 32])


def _wid():
    return lax.axis_index("s") * NC + lax.axis_index("c")


_NCH_G = EPW // CH_G


def _gather_body(p_hbm, q_hbm, dst_hbm, src_hbm, h_hbm,
                 idxd_v, idxs_v, rp0_v, rq0_v, rp1_v, rq1_v, hb0_v, hb1_v,
                 sem0, sem1, semw0, semw1):
    base = _wid() * EPW
    pltpu.sync_copy(dst_hbm.at[pl.ds(base, EPW)], idxd_v)
    pltpu.sync_copy(src_hbm.at[pl.ds(base, EPW)], idxs_v)

    rbufs = ((rp0_v, rq0_v, hb0_v, sem0, semw0),
             (rp1_v, rq1_v, hb1_v, sem1, semw1))

    def issue(ci, rp, rq, sem):
        s = pl.ds(ci * CH_G, CH_G)
        pltpu.async_copy(p_hbm.at[idxd_v.at[s]], rp, sem)
        pltpu.async_copy(q_hbm.at[idxs_v.at[s]], rq, sem)

    def wait(rp, rq, sem):
        pltpu.make_async_copy(p_hbm.at[idxd_v.at[pl.ds(0, CH_G)]], rp,
                              sem).wait()
        pltpu.make_async_copy(q_hbm.at[idxs_v.at[pl.ds(0, CH_G)]], rq,
                              sem).wait()

    def wait_write(hb, wsem):
        pltpu.make_async_copy(
            hb, h_hbm.at[pl.ds(0, CH_G)], wsem).wait()

    issue(0, rbufs[0][0], rbufs[0][1], rbufs[0][3])

    def pair(i2, _):
        for b in range(2):
            ci = i2 * 2 + b
            rp, rq, hb, sem, wsem = rbufs[b]
            nrp, nrq, _nhb, nsem, _nwsem = rbufs[1 - b]

            @pl.when(ci + 1 < _NCH_G)
            def _():
                issue(ci + 1, nrp, nrq, nsem)

            wait(rp, rq, sem)

            @pl.when(ci >= 2)
            def _():
                wait_write(hb, wsem)

            def row(i, _):
                for c in range(2):
                    s0 = pl.ds(c * 32, 16)
                    s1 = pl.ds(c * 32 + 16, 16)
                    a = rp[i, s0] + rq[i, s0]
                    bb = rp[i, s1] + rq[i, s1]
                    hb[i, pl.ds(c * 32, 32)] = plsc.pack(
                        a, bb, format=plsc.PackFormat.INTERLEAVED)
                return 0

            lax.fori_loop(0, CH_G, row, 0, unroll=4)
            pltpu.async_copy(
                hb, h_hbm.at[pl.ds(base + ci * CH_G, CH_G)], wsem)
        return 0

    lax.fori_loop(0, _NCH_G // 2, pair, 0)
    wait_write(rbufs[0][2], rbufs[0][4])
    wait_write(rbufs[1][2], rbufs[1][4])


def _sc_gather(p, q, dst, src):
    return pl.kernel(
        _gather_body,
        out_type=jax.ShapeDtypeStruct((EPAD, HID), jnp.bfloat16),
        mesh=_MESH,
        scratch_types=[
            pltpu.VMEM((EPW,), jnp.int32),
            pltpu.VMEM((EPW,), jnp.int32),
            pltpu.VMEM((CH_G, HID), jnp.float32),
            pltpu.VMEM((CH_G, HID), jnp.float32),
            pltpu.VMEM((CH_G, HID), jnp.float32),
            pltpu.VMEM((CH_G, HID), jnp.float32),
            pltpu.VMEM((CH_G, HID), jnp.bfloat16),
            pltpu.VMEM((CH_G, HID), jnp.bfloat16),
            pltpu.SemaphoreType.DMA,
            pltpu.SemaphoreType.DMA,
            pltpu.SemaphoreType.DMA,
            pltpu.SemaphoreType.DMA,
        ],
        compiler_params=pltpu.CompilerParams(use_tc_tiling_on_sc=False, needs_layout_passes=False),
    )(p, q, dst, src)


_EBLK = 4096


def _mm_t_body(h_ref, w_ref, b_ref, o_ref):
    o_ref[...] = lax.dot_general(
        w_ref[...], jax.nn.relu(h_ref[...].astype(jnp.float32)),
        (((0,), (1,)), ((), ())),
        preferred_element_type=jnp.float32) + b_ref[...]


def _edge_mlp_t(h, w2, b2):
    ncols = w2.shape[1]
    return pl.pallas_call(
        _mm_t_body,
        grid=(EPAD // _EBLK,),
        in_specs=[
            pl.BlockSpec((_EBLK, HID), lambda i: (i, 0)),
            pl.BlockSpec((HID, ncols), lambda i: (0, 0)),
            pl.BlockSpec((ncols, 1), lambda i: (0, 0)),
        ],
        out_specs=pl.BlockSpec((ncols, _EBLK), lambda i: (0, i)),
        out_shape=jax.ShapeDtypeStruct((ncols, EPAD), jnp.float32),
    )(h, w2, b2[:, None])


_NCH_S = EPAD // CH_S
_NBANK = 4


def _scatter_body(mt_hbm, dst_hbm, out_hbm, idx0_v, vals0_v, idx1_v, vals1_v,
                  a00, a01, a02, a03, a10, a11, a12, a13, outbuf_v,
                  sem0, sem1, *, ncols, do_relu):
    w = _wid()
    npairs = ncols // 2
    banks = ((a00, a01, a02, a03), (a10, a11, a12, a13))

    def init(i, _):
        s = pl.ds(i * 16, 16)
        neg = jnp.full((16,), -jnp.inf, jnp.float32)
        for c in range(2):
            for k in range(_NBANK):
                banks[c][k][s] = neg
        return 0

    lax.fori_loop(0, NOUT // 16, init, 0)

    @pl.when(w < npairs)
    def _():
        bufs = ((idx0_v, vals0_v, sem0), (idx1_v, vals1_v, sem1))

        def issue(ci, idx_v, vals_v, sem):
            off = ci * CH_S
            pltpu.async_copy(dst_hbm.at[pl.ds(off, CH_S)], idx_v, sem)
            pltpu.async_copy(
                mt_hbm.at[pl.ds(2 * w, 2), pl.ds(off, CH_S)], vals_v, sem)

        def wait(idx_v, vals_v, sem):
            pltpu.make_async_copy(dst_hbm.at[pl.ds(0, CH_S)], idx_v,
                                  sem).wait()
            pltpu.make_async_copy(
                mt_hbm.at[pl.ds(0, 2), pl.ds(0, CH_S)], vals_v, sem).wait()

        issue(0, *bufs[0])

        def pair(i2, _):
            for b in range(2):
                ci = i2 * 2 + b
                idx_v, vals_v, sem = bufs[b]

                @pl.when(ci + 1 < _NCH_S)
                def _():
                    issue(ci + 1, *bufs[1 - b])

                wait(idx_v, vals_v, sem)

                def sblk(g4, pend):
                    gb = g4 * _NBANK
                    idxs = [idx_v[pl.ds((gb + j) * 16, 16)]
                            for j in range(_NBANK)]
                    chains = [(banks[c][j], idxs[j],
                               vals_v[c, pl.ds((gb + j) * 16, 16)])
                              for j in range(_NBANK) for c in range(2)]
                    for _ in range(2):
                        curs = [plsc.load_gather(acc, [idx])
                                for acc, idx, _v in chains]
                        for (acc, idx, v), cur in zip(chains, curs):
                            plsc.store_scatter(acc, [idx], v, mask=v > cur)
                    curs = [plsc.load_gather(acc, [idx])
                            for acc, idx, _v in chains]
                    for (_acc, _idx, v), cur in zip(chains, curs):
                        pend = pend | (v > cur)
                    return pend

                pend = lax.fori_loop(0, CH_S // 16 // _NBANK, sblk,
                                     jnp.zeros((16,), jnp.bool_), unroll=2)

                @pl.when(jnp.any(pend))
                def _():
                    def grp_slow(g3, _):
                        for j in range(_NBANK):
                            g = g3 * _NBANK + j
                            s = pl.ds(g * 16, 16)
                            idx = idx_v[s]
                            for c in range(2):
                                acc = banks[c][j]
                                v = vals_v[c, s]

                                def round_(_b):
                                    c3 = plsc.load_gather(acc, [idx])
                                    m3 = v > c3
                                    plsc.store_scatter(acc, [idx], v,
                                                       mask=m3)
                                    return jnp.any(m3)

                                lax.while_loop(lambda bb: bb, round_, True)
                        return 0

                    lax.fori_loop(0, CH_S // 16 // _NBANK, grp_slow, 0)
            return 0

        lax.fori_loop(0, _NCH_S // 2, pair, 0)

        def fin(i, _):
            s = pl.ds(i * 16, 16)
            for c in range(2):
                v = banks[c][0][s]
                for k in range(1, _NBANK):
                    v = jnp.maximum(v, banks[c][k][s])
                if do_relu:
                    v = jnp.maximum(v, 0.0)
                else:
                    v = jnp.where(v == -jnp.inf, 0.0, v)
                outbuf_v[c, s] = v
            return 0

        lax.fori_loop(0, NOUT // 16, fin, 0)
        pltpu.sync_copy(outbuf_v, out_hbm.at[pl.ds(2 * w, 2), :])


def _sc_scatter_max(mt, dst, ncols, do_relu):
    body = functools.partial(_scatter_body, ncols=ncols, do_relu=do_relu)
    return pl.kernel(
        body,
        out_type=jax.ShapeDtypeStruct((ncols, NOUT), jnp.float32),
        mesh=_MESH,
        scratch_types=[
            pltpu.VMEM((CH_S,), jnp.int32),
            pltpu.VMEM((2, CH_S), jnp.float32),
            pltpu.VMEM((CH_S,), jnp.int32),
            pltpu.VMEM((2, CH_S), jnp.float32),
        ] + [pltpu.VMEM((NOUT,), jnp.float32)] * 8 + [
            pltpu.VMEM((2, NOUT), jnp.float32),
            pltpu.SemaphoreType.DMA,
            pltpu.SemaphoreType.DMA,
        ],
        compiler_params=pltpu.CompilerParams(use_tc_tiling_on_sc=True, needs_layout_passes=False),
    )(mt, dst)


_NBLK = 2000


def _node1_body(om_ref, w0_ref, b0_ref, w1_ref, b1_ref, wp_ref, bp_ref,
                wq_ref, p_ref, q_ref):
    f = jax.nn.relu(om_ref[...] @ w0_ref[...] + b0_ref[...])
    f = f @ w1_ref[...] + b1_ref[...]
    p_ref[...] = f @ wp_ref[...] + bp_ref[...]
    q_ref[...] = f @ wq_ref[...]


def _node1(om_pad, w0p, b0, w1, b1, wp, bp, wq):
    return pl.pallas_call(
        _node1_body,
        out_shape=[
            jax.ShapeDtypeStruct((N, HID), jnp.float32),
            jax.ShapeDtypeStruct((N, HID), jnp.float32),
        ],
    )(om_pad, w0p, b0, w1, b1, wp, bp, wq)


def _node23_body(xt_ref, wp_ref, wq_ref, ssp_ref, ssq_ref, p_ref, q_ref):
    xb = xt_ref[...]
    p_ref[...] = lax.dot_general(
        xb, wp_ref[...], (((0,), (0,)), ((), ())),
        preferred_element_type=jnp.float32) + ssp_ref[...]
    q_ref[...] = lax.dot_general(
        xb, wq_ref[...], (((0,), (0,)), ((), ())),
        preferred_element_type=jnp.float32) + ssq_ref[...]


def _node23(xt, wp, wq, ssp, ssq):
    return pl.pallas_call(
        _node23_body,
        out_shape=[
            jax.ShapeDtypeStruct((N, HID), jnp.float32),
            jax.ShapeDtypeStruct((N, HID), jnp.float32),
        ],
    )(xt, wp, wq, ssp, ssq)


def _pad_nodes(x):
    return jnp.pad(x, ((0, NPAD - N), (0, 0)))


def kernel(omega, edge_index, t, num_objs, lin0_W, lin0_b, lin1_W, lin1_b,
           gfp_W, emb_W, emb_b, m1_W1, m1_b1, m1_W2, m1_b2, m2_W1, m2_b1,
           m2_W2, m2_b2, m3_W1, m3_b1, m3_W2, m3_b2):
    src = edge_index[0]
    dst = edge_index[1]
    src_p = jnp.concatenate([src, jnp.zeros((EPAD - E,), jnp.int32)])
    dst_p = jnp.concatenate([dst, jnp.full((EPAD - E,), N, jnp.int32)])

    ts = t[:, 0]
    x_proj = ts[:, None] * gfp_W[None, :] * 2.0 * np.pi
    gfp_out = jnp.concatenate([jnp.sin(x_proj), jnp.cos(x_proj)], axis=-1)
    x_sigma = jax.nn.relu(jax.nn.relu(gfp_out) @ emb_W + emb_b)

    def split_w(W1, f):
        W1a, W1b = W1[:f], W1[f:]
        return W1a - W1b, W1b

    def edge_conv(p, q, w2, b2, do_relu):
        ncols = w2.shape[1]
        h = _sc_gather(p, q, dst_p, src_p)
        mt = _edge_mlp_t(h, w2[_HPERM, :], b2)
        return _sc_scatter_max(mt, dst_p, ncols, do_relu)[:, :N]

    wp1, wq1 = split_w(m1_W1, HID)
    om_pad = jnp.pad(omega, ((0, 0), (0, 5)))
    w0p = jnp.pad(lin0_W, ((0, 5), (0, 0)))
    p1, q1 = _node1(om_pad, w0p, lin0_b[None, :], lin1_W, lin1_b[None, :],
                    wp1, m1_b1[None, :], wq1)
    x1t = edge_conv(_pad_nodes(p1), _pad_nodes(q1), m1_W2, m1_b2, True)

    wp2, wq2 = split_w(m2_W1, HID + 32)
    ssp2 = jnp.tile(x_sigma @ wp2[HID:], (N // B, 1)) + m2_b1
    ssq2 = jnp.tile(x_sigma @ wq2[HID:], (N // B, 1))
    p2, q2 = _node23(x1t, wp2[:HID], wq2[:HID], ssp2, ssq2)
    x2t = edge_conv(_pad_nodes(p2), _pad_nodes(q2), m2_W2, m2_b2, True)

    wp3, wq3 = split_w(m3_W1, HID + 32)
    ssp3 = jnp.tile(x_sigma @ wp3[HID:], (N // B, 1)) + m3_b1
    ssq3 = jnp.tile(x_sigma @ wq3[HID:], (N // B, 1))
    p3, q3 = _node23(x2t, wp3[:HID], wq3[:HID], ssp3, ssq3)
    w2_3 = jnp.pad(m3_W2, ((0, 0), (0, 5)))
    b2_3 = jnp.pad(m3_b2, (0, 5))
    x3t = edge_conv(_pad_nodes(p3), _pad_nodes(q3), w2_3, b2_3, False)

    x3 = x3t[:3, :].T
    t_rep = jnp.repeat(t, N // B, axis=0)
    std = jnp.sqrt((SIGMA ** (2.0 * t_rep) - 1.0) / (2.0 * LOG_SIGMA))
    return x3 / (std + 1e-07)


# speedup vs baseline: 1.0339x; 1.0339x over previous
import functools

import jax
import jax.numpy as jnp
import numpy as np
from jax import lax
from jax.experimental import pallas as pl
from jax.experimental.pallas import tpu as pltpu
from jax.experimental.pallas import tpu_sc as plsc

N = 10000
NPAD = 10016
B = 250
E = 390000
EPAD = 393216
HID = 64
NC = 2
NS = 16
NW = NC * NS
EPW = EPAD // NW
CH_G = 256
CH_S = 2048
NOUT = 10112
SIGMA = 25.0
LOG_SIGMA = 3.2188758248682006

_MESH = plsc.VectorSubcoreMesh(core_axis_name="c", subcore_axis_name="s",
                               num_cores=NC, num_subcores=NS)


def _wid():
    return lax.axis_index("s") * NC + lax.axis_index("c")


_NCH_G = EPW // CH_G


def _gather_body(p_hbm, q_hbm, dst_hbm, src_hbm, h_hbm,
                 idxd_v, idxs_v, rp0_v, rq0_v, rp1_v, rq1_v,
                 sem0, sem1, semw0, semw1):
    base = _wid() * EPW
    pltpu.sync_copy(dst_hbm.at[pl.ds(base, EPW)], idxd_v)
    pltpu.sync_copy(src_hbm.at[pl.ds(base, EPW)], idxs_v)

    rbufs = ((rp0_v, rq0_v, sem0, semw0), (rp1_v, rq1_v, sem1, semw1))

    def issue(ci, rp, rq, sem):
        s = pl.ds(ci * CH_G, CH_G)
        pltpu.async_copy(p_hbm.at[idxd_v.at[s]], rp, sem)
        pltpu.async_copy(q_hbm.at[idxs_v.at[s]], rq, sem)

    def wait(rp, rq, sem):
        pltpu.make_async_copy(p_hbm.at[idxd_v.at[pl.ds(0, CH_G)]], rp,
                              sem).wait()
        pltpu.make_async_copy(q_hbm.at[idxs_v.at[pl.ds(0, CH_G)]], rq,
                              sem).wait()

    def wait_write(rp, wsem):
        pltpu.make_async_copy(
            rp, h_hbm.at[pl.ds(0, CH_G)], wsem).wait()

    issue(0, *rbufs[0][:3])

    def pair(i2, _):
        for b in range(2):
            ci = i2 * 2 + b
            rp, rq, sem, wsem = rbufs[b]
            nrp, nrq, nsem, nwsem = rbufs[1 - b]

            @pl.when(ci + 1 < _NCH_G)
            def _():
                @pl.when(ci >= 1)
                def _():
                    wait_write(nrp, nwsem)

                issue(ci + 1, nrp, nrq, nsem)

            wait(rp, rq, sem)

            def row(i, _):
                for c in range(HID // 16):
                    s = pl.ds(c * 16, 16)
                    rp[i, s] = rp[i, s] + rq[i, s]
                return 0

            lax.fori_loop(0, CH_G, row, 0, unroll=4)
            pltpu.async_copy(
                rp, h_hbm.at[pl.ds(base + ci * CH_G, CH_G)], wsem)
        return 0

    lax.fori_loop(0, _NCH_G // 2, pair, 0)
    wait_write(rbufs[0][0], rbufs[0][3])
    wait_write(rbufs[1][0], rbufs[1][3])


def _sc_gather(p, q, dst, src):
    return pl.kernel(
        _gather_body,
        out_type=jax.ShapeDtypeStruct((EPAD, HID), jnp.float32),
        mesh=_MESH,
        scratch_types=[
            pltpu.VMEM((EPW,), jnp.int32),
            pltpu.VMEM((EPW,), jnp.int32),
            pltpu.VMEM((CH_G, HID), jnp.float32),
            pltpu.VMEM((CH_G, HID), jnp.float32),
            pltpu.VMEM((CH_G, HID), jnp.float32),
            pltpu.VMEM((CH_G, HID), jnp.float32),
            pltpu.SemaphoreType.DMA,
            pltpu.SemaphoreType.DMA,
            pltpu.SemaphoreType.DMA,
            pltpu.SemaphoreType.DMA,
        ],
        compiler_params=pltpu.CompilerParams(use_tc_tiling_on_sc=False, needs_layout_passes=False),
    )(p, q, dst, src)


_EBLK = 4096


def _mm_t_body(h_ref, w_ref, b_ref, o_ref):
    o_ref[...] = lax.dot_general(
        w_ref[...], jax.nn.relu(h_ref[...]),
        (((0,), (1,)), ((), ())),
        preferred_element_type=jnp.float32) + b_ref[...]


def _edge_mlp_t(h, w2, b2):
    ncols = w2.shape[1]
    return pl.pallas_call(
        _mm_t_body,
        grid=(EPAD // _EBLK,),
        in_specs=[
            pl.BlockSpec((_EBLK, HID), lambda i: (i, 0)),
            pl.BlockSpec((HID, ncols), lambda i: (0, 0)),
            pl.BlockSpec((ncols, 1), lambda i: (0, 0)),
        ],
        out_specs=pl.BlockSpec((ncols, _EBLK), lambda i: (0, i)),
        out_shape=jax.ShapeDtypeStruct((ncols, EPAD), jnp.float32),
    )(h, w2, b2[:, None])


_NCH_S = EPAD // CH_S
_NBANK = 4


def _scatter_body(mt_hbm, dst_hbm, out_hbm, idx0_v, vals0_v, idx1_v, vals1_v,
                  a00, a01, a02, a03, a10, a11, a12, a13, outbuf_v,
                  sem0, sem1, *, ncols, do_relu):
    w = _wid()
    npairs = ncols // 2
    banks = ((a00, a01, a02, a03), (a10, a11, a12, a13))

    def init(i, _):
        s = pl.ds(i * 16, 16)
        neg = jnp.full((16,), -jnp.inf, jnp.float32)
        for c in range(2):
            for k in range(_NBANK):
                banks[c][k][s] = neg
        return 0

    lax.fori_loop(0, NOUT // 16, init, 0)

    @pl.when(w < npairs)
    def _():
        bufs = ((idx0_v, vals0_v, sem0), (idx1_v, vals1_v, sem1))

        def issue(ci, idx_v, vals_v, sem):
            off = ci * CH_S
            pltpu.async_copy(dst_hbm.at[pl.ds(off, CH_S)], idx_v, sem)
            pltpu.async_copy(
                mt_hbm.at[pl.ds(2 * w, 2), pl.ds(off, CH_S)], vals_v, sem)

        def wait(idx_v, vals_v, sem):
            pltpu.make_async_copy(dst_hbm.at[pl.ds(0, CH_S)], idx_v,
                                  sem).wait()
            pltpu.make_async_copy(
                mt_hbm.at[pl.ds(0, 2), pl.ds(0, CH_S)], vals_v, sem).wait()

        issue(0, *bufs[0])

        def pair(i2, _):
            for b in range(2):
                ci = i2 * 2 + b
                idx_v, vals_v, sem = bufs[b]

                @pl.when(ci + 1 < _NCH_S)
                def _():
                    issue(ci + 1, *bufs[1 - b])

                wait(idx_v, vals_v, sem)

                def sblk(g4, pend):
                    gb = g4 * _NBANK
                    idxs = [idx_v[pl.ds((gb + j) * 16, 16)]
                            for j in range(_NBANK)]
                    chains = [(banks[c][j], idxs[j],
                               vals_v[c, pl.ds((gb + j) * 16, 16)])
                              for j in range(_NBANK) for c in range(2)]
                    for _ in range(2):
                        curs = [plsc.load_gather(acc, [idx])
                                for acc, idx, _v in chains]
                        for (acc, idx, v), cur in zip(chains, curs):
                            plsc.store_scatter(acc, [idx], v, mask=v > cur)
                    curs = [plsc.load_gather(acc, [idx])
                            for acc, idx, _v in chains]
                    for (_acc, _idx, v), cur in zip(chains, curs):
                        pend = pend | (v > cur)
                    return pend

                pend = lax.fori_loop(0, CH_S // 16 // _NBANK, sblk,
                                     jnp.zeros((16,), jnp.bool_), unroll=2)

                @pl.when(jnp.any(pend))
                def _():
                    def grp_slow(g3, _):
                        for j in range(_NBANK):
                            g = g3 * _NBANK + j
                            s = pl.ds(g * 16, 16)
                            idx = idx_v[s]
                            for c in range(2):
                                acc = banks[c][j]
                                v = vals_v[c, s]

                                def round_(_b):
                                    c3 = plsc.load_gather(acc, [idx])
                                    m3 = v > c3
                                    plsc.store_scatter(acc, [idx], v,
                                                       mask=m3)
                                    return jnp.any(m3)

                                lax.while_loop(lambda bb: bb, round_, True)
                        return 0

                    lax.fori_loop(0, CH_S // 16 // _NBANK, grp_slow, 0)
            return 0

        lax.fori_loop(0, _NCH_S // 2, pair, 0)

        def fin(i, _):
            s = pl.ds(i * 16, 16)
            for c in range(2):
                v = banks[c][0][s]
                for k in range(1, _NBANK):
                    v = jnp.maximum(v, banks[c][k][s])
                if do_relu:
                    v = jnp.maximum(v, 0.0)
                else:
                    v = jnp.where(v == -jnp.inf, 0.0, v)
                outbuf_v[c, s] = v
            return 0

        lax.fori_loop(0, NOUT // 16, fin, 0)
        pltpu.sync_copy(outbuf_v, out_hbm.at[pl.ds(2 * w, 2), :])


def _sc_scatter_max(mt, dst, ncols, do_relu):
    body = functools.partial(_scatter_body, ncols=ncols, do_relu=do_relu)
    return pl.kernel(
        body,
        out_type=jax.ShapeDtypeStruct((ncols, NOUT), jnp.float32),
        mesh=_MESH,
        scratch_types=[
            pltpu.VMEM((CH_S,), jnp.int32),
            pltpu.VMEM((2, CH_S), jnp.float32),
            pltpu.VMEM((CH_S,), jnp.int32),
            pltpu.VMEM((2, CH_S), jnp.float32),
        ] + [pltpu.VMEM((NOUT,), jnp.float32)] * 8 + [
            pltpu.VMEM((2, NOUT), jnp.float32),
            pltpu.SemaphoreType.DMA,
            pltpu.SemaphoreType.DMA,
        ],
        compiler_params=pltpu.CompilerParams(use_tc_tiling_on_sc=True, needs_layout_passes=False),
    )(mt, dst)


_NBLK = 2000


def _node1_body(om_ref, w0_ref, b0_ref, w1_ref, b1_ref, wp_ref, bp_ref,
                wq_ref, p_ref, q_ref):
    f = jax.nn.relu(om_ref[...] @ w0_ref[...] + b0_ref[...])
    f = f @ w1_ref[...] + b1_ref[...]
    p_ref[...] = f @ wp_ref[...] + bp_ref[...]
    q_ref[...] = f @ wq_ref[...]


def _node1(om_pad, w0p, b0, w1, b1, wp, bp, wq):
    return pl.pallas_call(
        _node1_body,
        out_shape=[
            jax.ShapeDtypeStruct((N, HID), jnp.float32),
            jax.ShapeDtypeStruct((N, HID), jnp.float32),
        ],
    )(om_pad, w0p, b0, w1, b1, wp, bp, wq)


def _node23_body(xt_ref, wp_ref, wq_ref, ssp_ref, ssq_ref, p_ref, q_ref):
    xb = xt_ref[...]
    p_ref[...] = lax.dot_general(
        xb, wp_ref[...], (((0,), (0,)), ((), ())),
        preferred_element_type=jnp.float32) + ssp_ref[...]
    q_ref[...] = lax.dot_general(
        xb, wq_ref[...], (((0,), (0,)), ((), ())),
        preferred_element_type=jnp.float32) + ssq_ref[...]


def _node23(xt, wp, wq, ssp, ssq):
    return pl.pallas_call(
        _node23_body,
        out_shape=[
            jax.ShapeDtypeStruct((N, HID), jnp.float32),
            jax.ShapeDtypeStruct((N, HID), jnp.float32),
        ],
    )(xt, wp, wq, ssp, ssq)


def _pad_nodes(x):
    return jnp.pad(x, ((0, NPAD - N), (0, 0)))


def kernel(omega, edge_index, t, num_objs, lin0_W, lin0_b, lin1_W, lin1_b,
           gfp_W, emb_W, emb_b, m1_W1, m1_b1, m1_W2, m1_b2, m2_W1, m2_b1,
           m2_W2, m2_b2, m3_W1, m3_b1, m3_W2, m3_b2):
    src = edge_index[0]
    dst = edge_index[1]
    src_p = jnp.concatenate([src, jnp.zeros((EPAD - E,), jnp.int32)])
    dst_p = jnp.concatenate([dst, jnp.full((EPAD - E,), N, jnp.int32)])

    ts = t[:, 0]
    x_proj = ts[:, None] * gfp_W[None, :] * 2.0 * np.pi
    gfp_out = jnp.concatenate([jnp.sin(x_proj), jnp.cos(x_proj)], axis=-1)
    x_sigma = jax.nn.relu(jax.nn.relu(gfp_out) @ emb_W + emb_b)

    def split_w(W1, f):
        W1a, W1b = W1[:f], W1[f:]
        return W1a - W1b, W1b

    def edge_conv(p, q, w2, b2, do_relu):
        ncols = w2.shape[1]
        h = _sc_gather(p, q, dst_p, src_p)
        mt = _edge_mlp_t(h, w2, b2)
        return _sc_scatter_max(mt, dst_p, ncols, do_relu)[:, :N]

    wp1, wq1 = split_w(m1_W1, HID)
    om_pad = jnp.pad(omega, ((0, 0), (0, 5)))
    w0p = jnp.pad(lin0_W, ((0, 5), (0, 0)))
    p1, q1 = _node1(om_pad, w0p, lin0_b[None, :], lin1_W, lin1_b[None, :],
                    wp1, m1_b1[None, :], wq1)
    x1t = edge_conv(_pad_nodes(p1), _pad_nodes(q1), m1_W2, m1_b2, True)

    wp2, wq2 = split_w(m2_W1, HID + 32)
    ssp2 = jnp.tile(x_sigma @ wp2[HID:], (N // B, 1)) + m2_b1
    ssq2 = jnp.tile(x_sigma @ wq2[HID:], (N // B, 1))
    p2, q2 = _node23(x1t, wp2[:HID], wq2[:HID], ssp2, ssq2)
    x2t = edge_conv(_pad_nodes(p2), _pad_nodes(q2), m2_W2, m2_b2, True)

    wp3, wq3 = split_w(m3_W1, HID + 32)
    ssp3 = jnp.tile(x_sigma @ wp3[HID:], (N // B, 1)) + m3_b1
    ssq3 = jnp.tile(x_sigma @ wq3[HID:], (N // B, 1))
    p3, q3 = _node23(x2t, wp3[:HID], wq3[:HID], ssp3, ssq3)
    w2_3 = jnp.pad(m3_W2, ((0, 0), (0, 5)))
    b2_3 = jnp.pad(m3_b2, (0, 5))
    x3t = edge_conv(_pad_nodes(p3), _pad_nodes(q3), w2_3, b2_3, False)

    x3 = x3t[:3, :].T
    t_rep = jnp.repeat(t, N // B, axis=0)
    std = jnp.sqrt((SIGMA ** (2.0 * t_rep) - 1.0) / (2.0 * LOG_SIGMA))
    return x3 / (std + 1e-07)
